# Initial kernel scaffold; baseline (speedup 1.0000x reference)
import jax
import jax.numpy as jnp
from jax.experimental import pallas as pl


def kernel(x, edge_index, W0, b0, W1, b1, W2, b2, centroids):
    raise NotImplementedError("write your pallas kernel here")


# speedup vs baseline: 12.6110x; 12.6110x over previous
import dataclasses
import functools

import jax
import jax.numpy as jnp
import numpy as np
from jax import lax
from jax.experimental import pallas as pl
from jax.experimental.pallas import tpu as pltpu
from jax.experimental.pallas import tpu_sc as plsc

N = 10000
E = 320000
HID = 128
D = 4
SUB = HID // D
K = 4
K1 = 2
NUM_KS = K1 * ((1 << K) - 1)
GROUP_STARTS = [0, 2, 6, 14]
GROUP_SIZES = [2, 4, 8, 16]

NW = 32
EPW = E // NW
CH = 128
NCHUNK = E // CH
ITERS = (NCHUNK + NW - 1) // NW
NPT = N // 16

@functools.cache
def _mesh():
    return plsc.VectorSubcoreMesh(core_axis_name="c", subcore_axis_name="s")


_SC_PARAMS = pltpu.CompilerParams()
if "needs_layout_passes" in pltpu.CompilerParams.__dataclass_fields__:
    _SC_PARAMS = dataclasses.replace(_SC_PARAMS, needs_layout_passes=False)


def _deg_body(dst_hbm, cnt_hbm, idx_v, cnt_v):
    c = lax.axis_index("c")
    s = lax.axis_index("s")
    wid = s * 2 + c
    zero16 = jnp.zeros((16,), jnp.float32)
    one16 = jnp.ones((16,), jnp.float32)

    @pl.loop(0, N // 16)
    def _(i):
        cnt_v[pl.ds(i * 16, 16)] = zero16

    pltpu.sync_copy(dst_hbm.at[pl.ds(wid * EPW, EPW)], idx_v)

    @pl.loop(0, EPW // 16)
    def _(j):
        iv = idx_v[pl.ds(j * 16, 16)]
        plsc.addupdate_scatter(cnt_v, [iv], one16)

    pltpu.sync_copy(cnt_v, cnt_hbm.at[wid, 0])


@jax.jit
def _sc_degree(dst):
    f = pl.kernel(
        _deg_body,
        out_type=jax.ShapeDtypeStruct((NW, 1, N), jnp.float32),
        mesh=_mesh(),
        scratch_types=[
            pltpu.VMEM((EPW,), jnp.int32),
            pltpu.VMEM((N,), jnp.float32),
        ],
        compiler_params=_SC_PARAMS,
    )
    return f(dst)


def _prop_body(hs_hbm, src_hbm, dst_hbm, part_hbm, idx_s, idx_d, rows, acc_sh):
    c = lax.axis_index("c")
    s = lax.axis_index("s")
    wid = s * 2 + c
    zero16 = jnp.zeros((16,), jnp.float32)

    @pl.loop(0, CH)
    def _(i):
        @pl.loop(0, HID // 16)
        def _(j):
            rows[i, pl.ds(j * 16, 16)] = zero16

    base = s * 624

    @pl.loop(0, 6)
    def _(k):
        pltpu.sync_copy(rows.at[pl.ds(0, 104)], acc_sh.at[pl.ds(base + k * 104, 104)])

    @pl.when(s == 15)
    def _():
        pltpu.sync_copy(rows.at[pl.ds(0, 16)], acc_sh.at[pl.ds(16 * 624, 16)])

    plsc.subcore_barrier()

    @pl.loop(0, ITERS)
    def _(i):
        ch = i * NW + wid

        @pl.when(ch < NCHUNK)
        def _():
            off = ch * CH
            pltpu.sync_copy(src_hbm.at[pl.ds(off, CH)], idx_s)
            pltpu.sync_copy(dst_hbm.at[pl.ds(off, CH)], idx_d)
            pltpu.sync_copy(hs_hbm.at[idx_s], rows)
            pltpu.sync_copy(rows, acc_sh.at[idx_d], add=True)

    plsc.subcore_barrier()
    pltpu.sync_copy(acc_sh.at[pl.ds(base, 624)], part_hbm.at[c, pl.ds(base, 624)])

    @pl.when(s == 15)
    def _():
        pltpu.sync_copy(acc_sh.at[pl.ds(16 * 624, 16)],
                        part_hbm.at[c, pl.ds(16 * 624, 16)])


@jax.jit
def _sc_propagate(hs, src, dst):
    f = pl.kernel(
        _prop_body,
        out_type=jax.ShapeDtypeStruct((2, N, HID), jnp.float32),
        mesh=_mesh(),
        scratch_types=[
            pltpu.VMEM((CH,), jnp.int32),
            pltpu.VMEM((CH,), jnp.int32),
            pltpu.VMEM((CH, HID), jnp.float32),
            pltpu.VMEM_SHARED((N, HID), jnp.float32),
        ],
    )
    return f(hs, src, dst)


BN = 2000
GRID = N // BN

_HI = jax.lax.Precision.HIGHEST


def _pre_body(cnt_ref, x_ref, w_ref, dinv_ref, h_ref):
    deg = jnp.sum(cnt_ref[...], axis=1) + 1.0
    dinv = lax.rsqrt(deg)
    dinv2 = dinv[:, None]
    dinv_ref[...] = dinv2
    h = jnp.dot(x_ref[...], w_ref[...], preferred_element_type=jnp.float32,
                precision=_HI)
    h_ref[...] = dinv2 * h


@jax.jit
def _tc_pre(cnt, x, w0):
    return pl.pallas_call(
        _pre_body,
        grid=(GRID,),
        in_specs=[
            pl.BlockSpec((BN, NW), lambda i: (i, 0)),
            pl.BlockSpec((BN, HID), lambda i: (i, 0)),
            pl.BlockSpec((HID, HID), lambda i: (0, 0)),
        ],
        out_specs=[
            pl.BlockSpec((BN, 1), lambda i: (i, 0)),
            pl.BlockSpec((BN, HID), lambda i: (i, 0)),
        ],
        out_shape=[
            jax.ShapeDtypeStruct((N, 1), jnp.float32),
            jax.ShapeDtypeStruct((N, HID), jnp.float32),
        ],
    )(cnt, x, w0)


def _mid_body(p0_ref, p1_ref, hp_ref, dinv_ref, b_ref, w_ref, out_ref):
    dinv = dinv_ref[...]
    t = dinv * (p0_ref[...] + p1_ref[...] + hp_ref[...]) + b_ref[...]
    h = jnp.where(t >= 0, t, 0.5 * t)
    out_ref[...] = dinv * jnp.dot(h, w_ref[...], preferred_element_type=jnp.float32,
                                  precision=_HI)


@jax.jit
def _tc_mid(p0, p1, hp, dinv, b, w):
    return pl.pallas_call(
        _mid_body,
        grid=(GRID,),
        in_specs=[
            pl.BlockSpec((BN, HID), lambda i: (i, 0)),
            pl.BlockSpec((BN, HID), lambda i: (i, 0)),
            pl.BlockSpec((BN, HID), lambda i: (i, 0)),
            pl.BlockSpec((BN, 1), lambda i: (i, 0)),
            pl.BlockSpec((1, HID), lambda i: (0, 0)),
            pl.BlockSpec((HID, HID), lambda i: (0, 0)),
        ],
        out_specs=pl.BlockSpec((BN, HID), lambda i: (i, 0)),
        out_shape=jax.ShapeDtypeStruct((N, HID), jnp.float32),
    )(p0, p1, hp, dinv, b, w)


def _quant_body(q0_ref, q1_ref, hp_ref, dinv_ref, b_ref, cexp_ref, cr_ref,
                w_ref, out_ref, sp_ref):
    dinv = dinv_ref[...]
    t = dinv * (q0_ref[...] + q1_ref[...] + hp_ref[...]) + b_ref[...]
    h2 = jnp.where(t >= 0, t, 0.5 * t)
    logits = jnp.dot(h2, cexp_ref[...], preferred_element_type=jnp.float32,
                     precision=_HI)

    @pl.when(pl.program_id(0) == 0)
    def _():
        sp_ref[...] = jnp.zeros_like(sp_ref)

    pred_parts = []
    for d in range(D):
        feats = []
        for k in range(K):
            g0 = GROUP_STARTS[k]
            sz = GROUP_SIZES[k]
            sl = logits[:, d * NUM_KS + g0: d * NUM_KS + g0 + sz]
            gmax = jnp.max(sl, axis=1, keepdims=True)
            e = jnp.exp(sl - gmax)
            gs = jnp.sum(e, axis=1, keepdims=True)
            probs = e / gs
            sp_ref[d, g0:g0 + sz] += jnp.sum(probs, axis=0)
            sel = (sl == gmax).astype(jnp.float32)
            feats.append(jnp.dot(sel, cr_ref[d, g0:g0 + sz, :],
                                 preferred_element_type=jnp.float32,
                                 precision=_HI))
        pred_d = jnp.maximum(jnp.maximum(feats[0], feats[1]),
                             jnp.maximum(feats[2], feats[3]))
        pred_parts.append(pred_d)
    pred = jnp.concatenate(pred_parts, axis=1)
    h3 = h2 + pred
    out_ref[...] = dinv * jnp.dot(h3, w_ref[...], preferred_element_type=jnp.float32,
                                  precision=_HI)


@jax.jit
def _tc_quant(q0, q1, hp, dinv, b, cexp, cr, w):
    return pl.pallas_call(
        _quant_body,
        grid=(GRID,),
        in_specs=[
            pl.BlockSpec((BN, HID), lambda i: (i, 0)),
            pl.BlockSpec((BN, HID), lambda i: (i, 0)),
            pl.BlockSpec((BN, HID), lambda i: (i, 0)),
            pl.BlockSpec((BN, 1), lambda i: (i, 0)),
            pl.BlockSpec((1, HID), lambda i: (0, 0)),
            pl.BlockSpec((HID, D * NUM_KS), lambda i: (0, 0)),
            pl.BlockSpec((D, NUM_KS, SUB), lambda i: (0, 0, 0)),
            pl.BlockSpec((HID, HID), lambda i: (0, 0)),
        ],
        out_specs=[
            pl.BlockSpec((BN, HID), lambda i: (i, 0)),
            pl.BlockSpec((D, NUM_KS), lambda i: (0, 0)),
        ],
        out_shape=[
            jax.ShapeDtypeStruct((N, HID), jnp.float32),
            jax.ShapeDtypeStruct((D, NUM_KS), jnp.float32),
        ],
    )(q0, q1, hp, dinv, b, cexp, cr, w)


def _fin_body(r0_ref, r1_ref, hp_ref, dinv_ref, b_ref, sp_ref, tgt_ref,
              out_ref, reg_ref):
    dinv = dinv_ref[...]
    out_ref[...] = dinv * (r0_ref[...] + r1_ref[...] + hp_ref[...]) + b_ref[...]

    @pl.when(pl.program_id(0) == 0)
    def _():
        diff = sp_ref[...] * (1.0 / N) - tgt_ref[...]
        reg_ref[...] = jnp.sqrt(jnp.sum(diff * diff)).reshape(1, 1)


@jax.jit
def _tc_fin(r0, r1, hp, dinv, b, sp, tgt):
    return pl.pallas_call(
        _fin_body,
        grid=(GRID,),
        in_specs=[
            pl.BlockSpec((BN, HID), lambda i: (i, 0)),
            pl.BlockSpec((BN, HID), lambda i: (i, 0)),
            pl.BlockSpec((BN, HID), lambda i: (i, 0)),
            pl.BlockSpec((BN, 1), lambda i: (i, 0)),
            pl.BlockSpec((1, HID), lambda i: (0, 0)),
            pl.BlockSpec((D, NUM_KS), lambda i: (0, 0)),
            pl.BlockSpec((D, NUM_KS), lambda i: (0, 0)),
        ],
        out_specs=[
            pl.BlockSpec((BN, HID), lambda i: (i, 0)),
            pl.BlockSpec((1, 1), lambda i: (0, 0)),
        ],
        out_shape=[
            jax.ShapeDtypeStruct((N, HID), jnp.float32),
            jax.ShapeDtypeStruct((1, 1), jnp.float32),
        ],
    )(r0, r1, hp, dinv, b, sp, tgt)


def kernel(x, edge_index, W0, b0, W1, b1, W2, b2, centroids):
    src, dst = edge_index[0], edge_index[1]
    cnt = _sc_degree(dst)
    dinv, hs0 = _tc_pre(cnt.reshape(NW, N).T, x, W0)

    p = _sc_propagate(hs0, src, dst)
    hs1 = _tc_mid(p[0], p[1], hs0, dinv, b0.reshape(1, HID), W1)

    q = _sc_propagate(hs1, src, dst)

    cr = centroids.reshape(NUM_KS, D, SUB)
    cexp = jnp.einsum("mds,de->dsem", cr, jnp.eye(D, dtype=jnp.float32))
    cexp = cexp.reshape(HID, D * NUM_KS)
    cr2 = cr.transpose(1, 0, 2)
    k_blns = np.concatenate(
        [np.full(K1 << j, 1.0 / (K1 << j)) for j in range(K)]
    ).astype(np.float32)
    tgt = jnp.tile(jnp.asarray(k_blns), (D, 1))

    hs2, sp = _tc_quant(q[0], q[1], hs1, dinv, b1.reshape(1, HID), cexp, cr2, W2)

    r = _sc_propagate(hs2, src, dst)
    out, reg = _tc_fin(r[0], r[1], hs2, dinv, b2.reshape(1, HID), sp, tgt)
    return out, reg.reshape(())


# speedup vs baseline: 18.8531x; 1.4950x over previous
import dataclasses
import functools

import jax
import jax.numpy as jnp
import numpy as np
from jax import lax
from jax.experimental import pallas as pl
from jax.experimental.pallas import tpu as pltpu
from jax.experimental.pallas import tpu_sc as plsc

N = 10000
E = 320000
HID = 128
D = 4
SUB = HID // D
K = 4
K1 = 2
NUM_KS = K1 * ((1 << K) - 1)
GROUP_STARTS = [0, 2, 6, 14]
GROUP_SIZES = [2, 4, 8, 16]

NW = 32
EPW = E // NW
CH = 128
NCHUNK = E // CH
ITERS = (NCHUNK + NW - 1) // NW
NPT = N // 16

@functools.cache
def _mesh():
    return plsc.VectorSubcoreMesh(core_axis_name="c", subcore_axis_name="s")


_SC_PARAMS = pltpu.CompilerParams()
if "needs_layout_passes" in pltpu.CompilerParams.__dataclass_fields__:
    _SC_PARAMS = dataclasses.replace(_SC_PARAMS, needs_layout_passes=False)


def _deg_body(dst_hbm, cnt_hbm, idx_v, cnt_v):
    c = lax.axis_index("c")
    s = lax.axis_index("s")
    wid = s * 2 + c
    zero16 = jnp.zeros((16,), jnp.float32)
    one16 = jnp.ones((16,), jnp.float32)

    @pl.loop(0, N // 16)
    def _(i):
        cnt_v[pl.ds(i * 16, 16)] = zero16

    pltpu.sync_copy(dst_hbm.at[pl.ds(wid * EPW, EPW)], idx_v)

    @pl.loop(0, EPW // 16)
    def _(j):
        iv = idx_v[pl.ds(j * 16, 16)]
        plsc.addupdate_scatter(cnt_v, [iv], one16)

    pltpu.sync_copy(cnt_v, cnt_hbm.at[wid, 0])


@jax.jit
def _sc_degree(dst):
    f = pl.kernel(
        _deg_body,
        out_type=jax.ShapeDtypeStruct((NW, 1, N), jnp.float32),
        mesh=_mesh(),
        scratch_types=[
            pltpu.VMEM((EPW,), jnp.int32),
            pltpu.VMEM((N,), jnp.float32),
        ],
        compiler_params=_SC_PARAMS,
    )
    return f(dst)


def _prop_body(hs_hbm, src_hbm, dst_hbm, part_hbm,
               idx_s0, idx_d0, idx_s1, idx_d1, rows0, rows1,
               isem0, isem1, gsem0, gsem1, ssem0, ssem1, acc_sh):
    c = lax.axis_index("c")
    s = lax.axis_index("s")
    wid = s * 2 + c
    zero16 = jnp.zeros((16,), jnp.float32)

    @pl.loop(0, CH)
    def _(i):
        @pl.loop(0, HID // 16)
        def _(j):
            rows0[i, pl.ds(j * 16, 16)] = zero16

    base = s * 624

    @pl.loop(0, 6)
    def _(k):
        pltpu.sync_copy(rows0.at[pl.ds(0, 104)], acc_sh.at[pl.ds(base + k * 104, 104)])

    @pl.when(s == 15)
    def _():
        pltpu.sync_copy(rows0.at[pl.ds(0, 16)], acc_sh.at[pl.ds(16 * 624, 16)])

    plsc.subcore_barrier()

    bufs = ((idx_s0, idx_d0, rows0, isem0, gsem0, ssem0),
            (idx_s1, idx_d1, rows1, isem1, gsem1, ssem1))

    def chunk_of(i):
        return i * NW + wid

    def start_idx(i, b):
        idx_s, idx_d, _, isem, _, _ = bufs[b]

        @pl.when(chunk_of(i) < NCHUNK)
        def _():
            off = chunk_of(i) * CH
            pltpu.async_copy(src_hbm.at[pl.ds(off, CH)], idx_s, isem)
            pltpu.async_copy(dst_hbm.at[pl.ds(off, CH)], idx_d, isem)

    def wait_idx(i, b):
        idx_s, idx_d, _, isem, _, _ = bufs[b]

        @pl.when(chunk_of(i) < NCHUNK)
        def _():
            off = chunk_of(i) * CH
            pltpu.make_async_copy(src_hbm.at[pl.ds(off, CH)], idx_s, isem).wait()
            pltpu.make_async_copy(dst_hbm.at[pl.ds(off, CH)], idx_d, isem).wait()

    def start_gather(i, b):
        idx_s, _, rows, _, gsem, _ = bufs[b]

        @pl.when(chunk_of(i) < NCHUNK)
        def _():
            pltpu.async_copy(hs_hbm.at[idx_s], rows, gsem)

    def wait_gather(i, b):
        idx_s, _, rows, _, gsem, _ = bufs[b]

        @pl.when(chunk_of(i) < NCHUNK)
        def _():
            pltpu.make_async_copy(hs_hbm.at[idx_s], rows, gsem).wait()

    def start_scatter(i, b):
        _, idx_d, rows, _, _, ssem = bufs[b]

        @pl.when(chunk_of(i) < NCHUNK)
        def _():
            pltpu.async_copy(rows, acc_sh.at[idx_d], ssem, add=True)

    def wait_scatter(i, b):
        _, idx_d, rows, _, _, ssem = bufs[b]

        @pl.when(chunk_of(i) < NCHUNK)
        def _():
            pltpu.make_async_copy(rows, acc_sh.at[idx_d], ssem).wait()

    start_idx(0, 0)
    start_idx(1, 1)
    wait_idx(0, 0)
    start_gather(0, 0)

    @pl.loop(0, ITERS + 1, step=2)
    def _(i):
        wait_gather(i, 0)
        start_scatter(i, 0)
        wait_idx(i + 1, 1)
        start_gather(i + 1, 1)
        wait_scatter(i, 0)
        start_idx(i + 2, 0)
        wait_gather(i + 1, 1)
        start_scatter(i + 1, 1)
        wait_idx(i + 2, 0)
        start_gather(i + 2, 0)
        wait_scatter(i + 1, 1)
        start_idx(i + 3, 1)

    plsc.subcore_barrier()
    pltpu.sync_copy(acc_sh.at[pl.ds(base, 624)], part_hbm.at[c, pl.ds(base, 624)])

    @pl.when(s == 15)
    def _():
        pltpu.sync_copy(acc_sh.at[pl.ds(16 * 624, 16)],
                        part_hbm.at[c, pl.ds(16 * 624, 16)])


@jax.jit
def _sc_propagate(hs, src, dst):
    f = pl.kernel(
        _prop_body,
        out_type=jax.ShapeDtypeStruct((2, N, HID), jnp.float32),
        mesh=_mesh(),
        scratch_types=[
            pltpu.VMEM((CH,), jnp.int32),
            pltpu.VMEM((CH,), jnp.int32),
            pltpu.VMEM((CH,), jnp.int32),
            pltpu.VMEM((CH,), jnp.int32),
            pltpu.VMEM((CH, HID), jnp.float32),
            pltpu.VMEM((CH, HID), jnp.float32),
            pltpu.SemaphoreType.DMA,
            pltpu.SemaphoreType.DMA,
            pltpu.SemaphoreType.DMA,
            pltpu.SemaphoreType.DMA,
            pltpu.SemaphoreType.DMA,
            pltpu.SemaphoreType.DMA,
            pltpu.VMEM_SHARED((N, HID), jnp.float32),
        ],
        compiler_params=_SC_PARAMS,
    )
    return f(hs, src, dst)


BN = 2000
GRID = N // BN

_HI = jax.lax.Precision.HIGHEST


def _pre_body(cnt_ref, x_ref, w_ref, dinv_ref, h_ref):
    deg = jnp.sum(cnt_ref[...], axis=1) + 1.0
    dinv = lax.rsqrt(deg)
    dinv2 = dinv[:, None]
    dinv_ref[...] = dinv2
    h = jnp.dot(x_ref[...], w_ref[...], preferred_element_type=jnp.float32,
                precision=_HI)
    h_ref[...] = dinv2 * h


@jax.jit
def _tc_pre(cnt, x, w0):
    return pl.pallas_call(
        _pre_body,
        grid=(GRID,),
        in_specs=[
            pl.BlockSpec((BN, NW), lambda i: (i, 0)),
            pl.BlockSpec((BN, HID), lambda i: (i, 0)),
            pl.BlockSpec((HID, HID), lambda i: (0, 0)),
        ],
        out_specs=[
            pl.BlockSpec((BN, 1), lambda i: (i, 0)),
            pl.BlockSpec((BN, HID), lambda i: (i, 0)),
        ],
        out_shape=[
            jax.ShapeDtypeStruct((N, 1), jnp.float32),
            jax.ShapeDtypeStruct((N, HID), jnp.float32),
        ],
    )(cnt, x, w0)


def _mid_body(p0_ref, p1_ref, hp_ref, dinv_ref, b_ref, w_ref, out_ref):
    dinv = dinv_ref[...]
    t = dinv * (p0_ref[...] + p1_ref[...] + hp_ref[...]) + b_ref[...]
    h = jnp.where(t >= 0, t, 0.5 * t)
    out_ref[...] = dinv * jnp.dot(h, w_ref[...], preferred_element_type=jnp.float32,
                                  precision=_HI)


@jax.jit
def _tc_mid(p0, p1, hp, dinv, b, w):
    return pl.pallas_call(
        _mid_body,
        grid=(GRID,),
        in_specs=[
            pl.BlockSpec((BN, HID), lambda i: (i, 0)),
            pl.BlockSpec((BN, HID), lambda i: (i, 0)),
            pl.BlockSpec((BN, HID), lambda i: (i, 0)),
            pl.BlockSpec((BN, 1), lambda i: (i, 0)),
            pl.BlockSpec((1, HID), lambda i: (0, 0)),
            pl.BlockSpec((HID, HID), lambda i: (0, 0)),
        ],
        out_specs=pl.BlockSpec((BN, HID), lambda i: (i, 0)),
        out_shape=jax.ShapeDtypeStruct((N, HID), jnp.float32),
    )(p0, p1, hp, dinv, b, w)


def _quant_body(q0_ref, q1_ref, hp_ref, dinv_ref, b_ref, cexp_ref, cr_ref,
                w_ref, out_ref, sp_ref):
    dinv = dinv_ref[...]
    t = dinv * (q0_ref[...] + q1_ref[...] + hp_ref[...]) + b_ref[...]
    h2 = jnp.where(t >= 0, t, 0.5 * t)
    logits = jnp.dot(h2, cexp_ref[...], preferred_element_type=jnp.float32,
                     precision=_HI)

    @pl.when(pl.program_id(0) == 0)
    def _():
        sp_ref[...] = jnp.zeros_like(sp_ref)

    pred_parts = []
    for d in range(D):
        feats = []
        for k in range(K):
            g0 = GROUP_STARTS[k]
            sz = GROUP_SIZES[k]
            sl = logits[:, d * NUM_KS + g0: d * NUM_KS + g0 + sz]
            gmax = jnp.max(sl, axis=1, keepdims=True)
            e = jnp.exp(sl - gmax)
            gs = jnp.sum(e, axis=1, keepdims=True)
            probs = e / gs
            sp_ref[d, g0:g0 + sz] += jnp.sum(probs, axis=0)
            sel = (sl == gmax).astype(jnp.float32)
            feats.append(jnp.dot(sel, cr_ref[d, g0:g0 + sz, :],
                                 preferred_element_type=jnp.float32,
                                 precision=_HI))
        pred_d = jnp.maximum(jnp.maximum(feats[0], feats[1]),
                             jnp.maximum(feats[2], feats[3]))
        pred_parts.append(pred_d)
    pred = jnp.concatenate(pred_parts, axis=1)
    h3 = h2 + pred
    out_ref[...] = dinv * jnp.dot(h3, w_ref[...], preferred_element_type=jnp.float32,
                                  precision=_HI)


@jax.jit
def _tc_quant(q0, q1, hp, dinv, b, cexp, cr, w):
    return pl.pallas_call(
        _quant_body,
        grid=(GRID,),
        in_specs=[
            pl.BlockSpec((BN, HID), lambda i: (i, 0)),
            pl.BlockSpec((BN, HID), lambda i: (i, 0)),
            pl.BlockSpec((BN, HID), lambda i: (i, 0)),
            pl.BlockSpec((BN, 1), lambda i: (i, 0)),
            pl.BlockSpec((1, HID), lambda i: (0, 0)),
            pl.BlockSpec((HID, D * NUM_KS), lambda i: (0, 0)),
            pl.BlockSpec((D, NUM_KS, SUB), lambda i: (0, 0, 0)),
            pl.BlockSpec((HID, HID), lambda i: (0, 0)),
        ],
        out_specs=[
            pl.BlockSpec((BN, HID), lambda i: (i, 0)),
            pl.BlockSpec((D, NUM_KS), lambda i: (0, 0)),
        ],
        out_shape=[
            jax.ShapeDtypeStruct((N, HID), jnp.float32),
            jax.ShapeDtypeStruct((D, NUM_KS), jnp.float32),
        ],
    )(q0, q1, hp, dinv, b, cexp, cr, w)


def _fin_body(r0_ref, r1_ref, hp_ref, dinv_ref, b_ref, sp_ref, tgt_ref,
              out_ref, reg_ref):
    dinv = dinv_ref[...]
    out_ref[...] = dinv * (r0_ref[...] + r1_ref[...] + hp_ref[...]) + b_ref[...]

    @pl.when(pl.program_id(0) == 0)
    def _():
        diff = sp_ref[...] * (1.0 / N) - tgt_ref[...]
        reg_ref[...] = jnp.sqrt(jnp.sum(diff * diff)).reshape(1, 1)


@jax.jit
def _tc_fin(r0, r1, hp, dinv, b, sp, tgt):
    return pl.pallas_call(
        _fin_body,
        grid=(GRID,),
        in_specs=[
            pl.BlockSpec((BN, HID), lambda i: (i, 0)),
            pl.BlockSpec((BN, HID), lambda i: (i, 0)),
            pl.BlockSpec((BN, HID), lambda i: (i, 0)),
            pl.BlockSpec((BN, 1), lambda i: (i, 0)),
            pl.BlockSpec((1, HID), lambda i: (0, 0)),
            pl.BlockSpec((D, NUM_KS), lambda i: (0, 0)),
            pl.BlockSpec((D, NUM_KS), lambda i: (0, 0)),
        ],
        out_specs=[
            pl.BlockSpec((BN, HID), lambda i: (i, 0)),
            pl.BlockSpec((1, 1), lambda i: (0, 0)),
        ],
        out_shape=[
            jax.ShapeDtypeStruct((N, HID), jnp.float32),
            jax.ShapeDtypeStruct((1, 1), jnp.float32),
        ],
    )(r0, r1, hp, dinv, b, sp, tgt)


def kernel(x, edge_index, W0, b0, W1, b1, W2, b2, centroids):
    src, dst = edge_index[0], edge_index[1]
    cnt = _sc_degree(dst)
    dinv, hs0 = _tc_pre(cnt.reshape(NW, N).T, x, W0)

    p = _sc_propagate(hs0, src, dst)
    hs1 = _tc_mid(p[0], p[1], hs0, dinv, b0.reshape(1, HID), W1)

    q = _sc_propagate(hs1, src, dst)

    cr = centroids.reshape(NUM_KS, D, SUB)
    cexp = jnp.einsum("mds,de->dsem", cr, jnp.eye(D, dtype=jnp.float32))
    cexp = cexp.reshape(HID, D * NUM_KS)
    cr2 = cr.transpose(1, 0, 2)
    k_blns = np.concatenate(
        [np.full(K1 << j, 1.0 / (K1 << j)) for j in range(K)]
    ).astype(np.float32)
    tgt = jnp.tile(jnp.asarray(k_blns), (D, 1))

    hs2, sp = _tc_quant(q[0], q[1], hs1, dinv, b1.reshape(1, HID), cexp, cr2, W2)

    r = _sc_propagate(hs2, src, dst)
    out, reg = _tc_fin(r[0], r[1], hs2, dinv, b2.reshape(1, HID), sp, tgt)
    return out, reg.reshape(())
